# baseline (speedup 1.0000x reference)
import functools

import jax
import jax.numpy as jnp
from jax import lax
from jax.experimental import pallas as pl
from jax.experimental.pallas import tpu as pltpu
from jax.experimental.pallas import tpu_sc as plsc

EMBEDDING_DIM = 64
HIDDEN_DIM = 64
HIDDEN_PAD = 128
VOCAB = 1000

_NUM_CORES = 2
_NUM_SUBCORES = 16
_NUM_WORKERS = _NUM_CORES * _NUM_SUBCORES

_CHUNK = 128
_L_BLOCK = 8
_B_BLOCK = 512


def _htable_body(emb_ref, wh_ref, bh_ref, out_ref):
    h = jnp.dot(emb_ref[...], wh_ref[...], preferred_element_type=jnp.float32)
    h = jnp.tanh(h + bh_ref[...])
    out_ref[...] = jnp.concatenate(
        [h, jnp.zeros((VOCAB, HIDDEN_PAD - HIDDEN_DIM), jnp.float32)], axis=1
    )


def _build_htable(emb_table, W_h, b_h):
    return pl.pallas_call(
        _htable_body,
        out_shape=jax.ShapeDtypeStruct((VOCAB, HIDDEN_PAD), jnp.float32),
    )(emb_table, W_h, b_h.reshape(1, HIDDEN_DIM))


def _gather_body(
    n_tokens, htab_hbm, idx_hbm, out_hbm, idx_v, buf0, buf1,
    sem_g0, sem_g1, sem_s0, sem_s1,
):
    b_per_w = n_tokens // _NUM_WORKERS
    n_chunks = b_per_w // _CHUNK
    cid = lax.axis_index("c")
    sid = lax.axis_index("s")
    wid = sid * _NUM_CORES + cid
    base = wid * b_per_w

    pltpu.sync_copy(idx_hbm.at[pl.ds(base, b_per_w)], idx_v)

    bufs = (buf0, buf1)
    gsems = (sem_g0, sem_g1)
    ssems = (sem_s0, sem_s1)

    def start_gather(t, p):
        pltpu.async_copy(
            htab_hbm.at[idx_v.at[pl.ds(t * _CHUNK, _CHUNK)]], bufs[p], gsems[p]
        )

    def start_scatter(t, p):
        pltpu.async_copy(
            bufs[p], out_hbm.at[pl.ds(base + t * _CHUNK, _CHUNK)], ssems[p]
        )

    def wait_gather(p):
        pltpu.make_async_copy(
            htab_hbm.at[pl.ds(0, _CHUNK)], bufs[p], gsems[p]
        ).wait()

    def wait_scatter(p):
        pltpu.make_async_copy(
            bufs[p], out_hbm.at[pl.ds(base, _CHUNK)], ssems[p]
        ).wait()

    start_gather(0, 0)

    def pair_body(i, carry):
        for p in (0, 1):
            t = 2 * i + p

            @pl.when(t + 1 < n_chunks)
            def _():
                @pl.when(t >= 1)
                def _():
                    wait_scatter(1 - p)

                start_gather(t + 1, 1 - p)

            wait_gather(p)
            start_scatter(t, p)
        return carry

    lax.fori_loop(0, n_chunks // 2, pair_body, 0)
    wait_scatter(0)
    wait_scatter(1)


def _gather_rows(htable, idx):
    n_tokens = idx.shape[0]
    b_per_w = n_tokens // _NUM_WORKERS
    mesh = plsc.VectorSubcoreMesh(core_axis_name="c", subcore_axis_name="s")
    return pl.kernel(
        functools.partial(_gather_body, n_tokens),
        out_type=jax.ShapeDtypeStruct((n_tokens, HIDDEN_PAD), jnp.float32),
        mesh=mesh,
        scratch_types=[
            pltpu.VMEM((b_per_w,), jnp.int32),
            pltpu.VMEM((_CHUNK, HIDDEN_PAD), jnp.float32),
            pltpu.VMEM((_CHUNK, HIDDEN_PAD), jnp.float32),
            pltpu.SemaphoreType.DMA,
            pltpu.SemaphoreType.DMA,
            pltpu.SemaphoreType.DMA,
            pltpu.SemaphoreType.DMA,
        ],
        compiler_params=pltpu.CompilerParams(use_tc_tiling_on_sc=True),
    )(htable, idx)


_K_PACK = 256


def _matmul_body(hg_ref, wpk_ref, out_ref):
    dn = (((1,), (0,)), ((), ()))
    ones2 = jnp.concatenate(
        [
            jnp.ones((2, _B_BLOCK), jnp.bfloat16),
            jnp.zeros((_K_PACK - 3 * HIDDEN_DIM - 2, _B_BLOCK), jnp.bfloat16),
        ],
        axis=0,
    )
    wpk = wpk_ref[...]
    for l in range(_L_BLOCK):
        v = hg_ref[l]
        vt = v.T[:HIDDEN_DIM]
        vt_hi = vt.astype(jnp.bfloat16)
        vt_lo = (vt - vt_hi.astype(jnp.float32)).astype(jnp.bfloat16)
        rhs = jnp.concatenate([vt_hi, vt_hi, vt_lo, ones2], axis=0)
        out_ref[l] = lax.dot_general(
            wpk, rhs, dn, preferred_element_type=jnp.float32
        )


def _output_matmul(hg, w_pack, L, B):
    hg3 = hg.reshape(L, B, HIDDEN_PAD)
    grid = (L // _L_BLOCK, B // _B_BLOCK)
    return pl.pallas_call(
        _matmul_body,
        grid=grid,
        in_specs=[
            pl.BlockSpec((_L_BLOCK, _B_BLOCK, HIDDEN_PAD), lambda i, j: (i, j, 0)),
            pl.BlockSpec((VOCAB, _K_PACK), lambda i, j: (0, 0)),
        ],
        out_specs=pl.BlockSpec((_L_BLOCK, VOCAB, _B_BLOCK), lambda i, j: (i, 0, j)),
        out_shape=jax.ShapeDtypeStruct((L, VOCAB, B), jnp.float32),
        compiler_params=pltpu.CompilerParams(
            dimension_semantics=("arbitrary", "arbitrary"),
        ),
    )(hg3, w_pack)


def kernel(x, emb_table, W_h, b_h, W_o, b_o):
    B, L = x.shape
    htable = _build_htable(emb_table, W_h, b_h)
    idx = x.T.reshape(-1).astype(jnp.int32)
    hg = _gather_rows(htable, idx)
    wt = W_o.T
    w_hi = wt.astype(jnp.bfloat16)
    w_lo = (wt - w_hi.astype(jnp.float32)).astype(jnp.bfloat16)
    b_hi = b_o.astype(jnp.bfloat16)
    b_lo = (b_o - b_hi.astype(jnp.float32)).astype(jnp.bfloat16)
    w_pack = jnp.concatenate(
        [
            w_hi,
            w_lo,
            w_hi,
            b_hi.reshape(VOCAB, 1),
            b_lo.reshape(VOCAB, 1),
            jnp.zeros((VOCAB, _K_PACK - 3 * HIDDEN_DIM - 2), jnp.bfloat16),
        ],
        axis=1,
    )
    out_t = _output_matmul(hg, w_pack, L, B)
    return jnp.transpose(out_t, (2, 0, 1))


# speedup vs baseline: 1.0163x; 1.0163x over previous
import functools

import jax
import jax.numpy as jnp
from jax import lax
from jax.experimental import pallas as pl
from jax.experimental.pallas import tpu as pltpu
from jax.experimental.pallas import tpu_sc as plsc

EMBEDDING_DIM = 64
HIDDEN_DIM = 64
HIDDEN_PAD = 128
VOCAB = 1000

_NUM_CORES = 2
_NUM_SUBCORES = 16
_NUM_WORKERS = _NUM_CORES * _NUM_SUBCORES

_CHUNK = 128
_L_BLOCK = 10
_B_BLOCK = 512
_L_HALF = 100


def _htable_body(emb_ref, wh_ref, bh_ref, out_ref):
    h = jnp.dot(emb_ref[...], wh_ref[...], preferred_element_type=jnp.float32)
    h = jnp.tanh(h + bh_ref[...])
    out_ref[...] = jnp.concatenate(
        [h, jnp.zeros((VOCAB, HIDDEN_PAD - HIDDEN_DIM), jnp.float32)], axis=1
    )


def _build_htable(emb_table, W_h, b_h):
    return pl.pallas_call(
        _htable_body,
        out_shape=jax.ShapeDtypeStruct((VOCAB, HIDDEN_PAD), jnp.float32),
    )(emb_table, W_h, b_h.reshape(1, HIDDEN_DIM))


def _gather_body(
    n_tokens, htab_hbm, idx_hbm, out_hbm, idx_v, buf0, buf1,
    sem_g0, sem_g1, sem_s0, sem_s1,
):
    b_per_w = n_tokens // _NUM_WORKERS
    n_chunks = b_per_w // _CHUNK
    cid = lax.axis_index("c")
    sid = lax.axis_index("s")
    wid = sid * _NUM_CORES + cid
    base = wid * b_per_w

    pltpu.sync_copy(idx_hbm.at[pl.ds(base, b_per_w)], idx_v)

    bufs = (buf0, buf1)
    gsems = (sem_g0, sem_g1)
    ssems = (sem_s0, sem_s1)

    def start_gather(t, p):
        pltpu.async_copy(
            htab_hbm.at[idx_v.at[pl.ds(t * _CHUNK, _CHUNK)]], bufs[p], gsems[p]
        )

    def start_scatter(t, p):
        pltpu.async_copy(
            bufs[p], out_hbm.at[pl.ds(base + t * _CHUNK, _CHUNK)], ssems[p]
        )

    def wait_gather(p):
        pltpu.make_async_copy(
            htab_hbm.at[pl.ds(0, _CHUNK)], bufs[p], gsems[p]
        ).wait()

    def wait_scatter(p):
        pltpu.make_async_copy(
            bufs[p], out_hbm.at[pl.ds(base, _CHUNK)], ssems[p]
        ).wait()

    start_gather(0, 0)

    def pair_body(i, carry):
        for p in (0, 1):
            t = 2 * i + p

            @pl.when(t + 1 < n_chunks)
            def _():
                @pl.when(t >= 1)
                def _():
                    wait_scatter(1 - p)

                start_gather(t + 1, 1 - p)

            wait_gather(p)
            start_scatter(t, p)
        return carry

    lax.fori_loop(0, n_chunks // 2, pair_body, 0)
    if n_chunks % 2:
        t = n_chunks - 1
        p = t % 2
        wait_gather(p)
        start_scatter(t, p)
    wait_scatter(0)
    wait_scatter(1)


def _gather_rows(htable, idx):
    n_tokens = idx.shape[0]
    b_per_w = n_tokens // _NUM_WORKERS
    mesh = plsc.VectorSubcoreMesh(core_axis_name="c", subcore_axis_name="s")
    return pl.kernel(
        functools.partial(_gather_body, n_tokens),
        out_type=jax.ShapeDtypeStruct((n_tokens, HIDDEN_PAD), jnp.float32),
        mesh=mesh,
        scratch_types=[
            pltpu.VMEM((b_per_w,), jnp.int32),
            pltpu.VMEM((_CHUNK, HIDDEN_PAD), jnp.float32),
            pltpu.VMEM((_CHUNK, HIDDEN_PAD), jnp.float32),
            pltpu.SemaphoreType.DMA,
            pltpu.SemaphoreType.DMA,
            pltpu.SemaphoreType.DMA,
            pltpu.SemaphoreType.DMA,
        ],
        compiler_params=pltpu.CompilerParams(use_tc_tiling_on_sc=True),
    )(htable, idx)


_K_PACK = 256


def _matmul_body(hg_ref, wpk_ref, out_ref):
    dn = (((1,), (0,)), ((), ()))
    ones2 = jnp.concatenate(
        [
            jnp.ones((2, _B_BLOCK), jnp.bfloat16),
            jnp.zeros((_K_PACK - 3 * HIDDEN_DIM - 2, _B_BLOCK), jnp.bfloat16),
        ],
        axis=0,
    )
    wpk = wpk_ref[...]
    for l in range(_L_BLOCK):
        v = hg_ref[l]
        vt = v.T[:HIDDEN_DIM]
        vt_hi = vt.astype(jnp.bfloat16)
        vt_lo = (vt - vt_hi.astype(jnp.float32)).astype(jnp.bfloat16)
        rhs = jnp.concatenate([vt_hi, vt_hi, vt_lo, ones2], axis=0)
        out_ref[l] = lax.dot_general(
            wpk, rhs, dn, preferred_element_type=jnp.float32
        )


def _matmul_body_alias(hg_ref, wpk_ref, prev_ref, out_ref):
    del prev_ref
    _matmul_body(hg_ref, wpk_ref, out_ref)


def _output_matmul_half(hg_half, w_pack, L, B, l_off, out_prev):
    hg3 = hg_half.reshape(_L_HALF, B, HIDDEN_PAD)
    grid = (_L_HALF // _L_BLOCK, B // _B_BLOCK)
    off = l_off // _L_BLOCK
    in_specs = [
        pl.BlockSpec((_L_BLOCK, _B_BLOCK, HIDDEN_PAD), lambda i, j: (i, j, 0)),
        pl.BlockSpec((VOCAB, _K_PACK), lambda i, j: (0, 0)),
    ]
    args = [hg3, w_pack]
    body = _matmul_body
    kwargs = {}
    if out_prev is not None:
        in_specs.append(pl.BlockSpec(memory_space=pl.ANY))
        args.append(out_prev)
        body = _matmul_body_alias
        kwargs["input_output_aliases"] = {2: 0}
    return pl.pallas_call(
        body,
        grid=grid,
        in_specs=in_specs,
        out_specs=pl.BlockSpec(
            (_L_BLOCK, VOCAB, _B_BLOCK), lambda i, j: (i + off, 0, j)
        ),
        out_shape=jax.ShapeDtypeStruct((L, VOCAB, B), jnp.float32),
        compiler_params=pltpu.CompilerParams(
            dimension_semantics=("arbitrary", "arbitrary"),
        ),
        **kwargs,
    )(*args)


def kernel(x, emb_table, W_h, b_h, W_o, b_o):
    B, L = x.shape
    htable = _build_htable(emb_table, W_h, b_h)
    xt = x.T.astype(jnp.int32)
    idx_a = xt[:_L_HALF].reshape(-1)
    idx_b = xt[_L_HALF:].reshape(-1)
    wt = W_o.T
    w_hi = wt.astype(jnp.bfloat16)
    w_lo = (wt - w_hi.astype(jnp.float32)).astype(jnp.bfloat16)
    b_hi = b_o.astype(jnp.bfloat16)
    b_lo = (b_o - b_hi.astype(jnp.float32)).astype(jnp.bfloat16)
    w_pack = jnp.concatenate(
        [
            w_hi,
            w_lo,
            w_hi,
            b_hi.reshape(VOCAB, 1),
            b_lo.reshape(VOCAB, 1),
            jnp.zeros((VOCAB, _K_PACK - 3 * HIDDEN_DIM - 2), jnp.bfloat16),
        ],
        axis=1,
    )
    hg_a = _gather_rows(htable, idx_a)
    hg_b = _gather_rows(htable, idx_b)
    out_t = _output_matmul_half(hg_a, w_pack, L, B, 0, None)
    out_t = _output_matmul_half(hg_b, w_pack, L, B, _L_HALF, out_t)
    return jnp.transpose(out_t, (2, 0, 1))
